# baseline (speedup 1.0000x reference)
import jax
import jax.numpy as jnp
from jax import lax
from jax.experimental import pallas as pl
from jax.experimental.pallas import tpu as pltpu

B = 1024
C = 100000
S = 32.0
M_MARGIN = 0.35
T_HARD = 0.2

_CB = 2000
_NSTEP = C // _CB

_SHIFT = S * ((T_HARD + 1.0) + T_HARD)
_LOG2E = 1.4426950408889634
_K2 = S * _LOG2E
_M2 = _SHIFT * _LOG2E


def _gt_body(x_ref, t_ref, gt_ref, acc):
    i = pl.program_id(0)

    @pl.when(i == 0)
    def _():
        acc[...] = jnp.zeros((1, B), jnp.float32)

    cio = i * _CB + lax.broadcasted_iota(jnp.int32, (_CB, B), 0)
    hit = cio == t_ref[...]
    acc[...] += jnp.sum(jnp.where(hit, x_ref[...], 0.0), axis=0, keepdims=True)

    @pl.when(i == _NSTEP - 1)
    def _():
        gt_ref[...] = acc[...]


def _ce_body(x_ref, g_ref, o_ref, acc):
    i = pl.program_id(0)

    @pl.when(i == 0)
    def _():
        acc[...] = jnp.zeros((1, B), jnp.float32)

    g = jnp.clip(g_ref[...], -1.0, 1.0)
    gm = g - M_MARGIN
    v = jnp.clip(x_ref[...], -1.0, 1.0)
    u = jnp.where(v > gm, (T_HARD + 1.0) * v + T_HARD, v)
    acc[...] += jnp.sum(jnp.exp2(u * _K2 - _M2), axis=0, keepdims=True)

    @pl.when(i == _NSTEP - 1)
    def _():
        fgt = jnp.where(g > 0.0, gm, g)
        trg = (T_HARD + 1.0) * g + T_HARD
        ssum = acc[...] - jnp.exp2(trg * _K2 - _M2) + jnp.exp2(fgt * _K2 - _M2)
        lse = jnp.log(ssum) + _SHIFT
        loss = jnp.sum(lse - S * fgt) * (1.0 / B)
        o_ref[...] = loss.reshape(1, 1)


def kernel(inputs, target):
    xt = inputs.T
    t2d = target.reshape(1, B)
    gt = pl.pallas_call(
        _gt_body,
        grid=(_NSTEP,),
        in_specs=[
            pl.BlockSpec((_CB, B), lambda i: (i, 0)),
            pl.BlockSpec((1, B), lambda i: (0, 0)),
        ],
        out_specs=pl.BlockSpec((1, B), lambda i: (0, 0)),
        out_shape=jax.ShapeDtypeStruct((1, B), jnp.float32),
        scratch_shapes=[pltpu.VMEM((1, B), jnp.float32)],
    )(xt, t2d)
    loss = pl.pallas_call(
        _ce_body,
        grid=(_NSTEP,),
        in_specs=[
            pl.BlockSpec((_CB, B), lambda i: (i, 0)),
            pl.BlockSpec((1, B), lambda i: (0, 0)),
        ],
        out_specs=pl.BlockSpec((1, 1), lambda i: (0, 0)),
        out_shape=jax.ShapeDtypeStruct((1, 1), jnp.float32),
        scratch_shapes=[pltpu.VMEM((1, B), jnp.float32)],
    )(xt, gt)
    return loss[0, 0]


# speedup vs baseline: 1.3724x; 1.3724x over previous
import functools

import jax
import jax.numpy as jnp
from jax import lax
from jax.experimental import pallas as pl
from jax.experimental.pallas import tpu as pltpu
from jax.experimental.pallas import tpu_sc as plsc

B = 1024
C = 100000
S = 32.0
M_MARGIN = 0.35
T_HARD = 0.2

_NC = 2
_NS = 16
_NW = _NC * _NS
_BPW = B // _NW


def _sc_body(xt_hbm, tgt_hbm, out_hbm, tgt_v, rows_v, sem):
    wid = lax.axis_index("s") * _NC + lax.axis_index("c")
    base = wid * _BPW
    pltpu.sync_copy(tgt_hbm.at[pl.ds(base, _BPW)], tgt_v)
    pltpu.async_copy(xt_hbm.at[tgt_v], rows_v, sem).wait()
    pltpu.sync_copy(rows_v, out_hbm.at[pl.ds(base, _BPW)])


def _sc_rowgather(xt, target):
    k = functools.partial(
        pl.kernel,
        out_type=jax.ShapeDtypeStruct((B, B), jnp.float32),
        mesh=plsc.VectorSubcoreMesh(
            core_axis_name="c", subcore_axis_name="s",
            num_cores=_NC, num_subcores=_NS,
        ),
        scratch_types=[
            pltpu.VMEM((_BPW,), jnp.int32),
            pltpu.VMEM((_BPW, B), jnp.float32),
            pltpu.SemaphoreType.DMA,
        ],
        compiler_params=pltpu.CompilerParams(use_tc_tiling_on_sc=True),
    )(_sc_body)
    return k(xt, target)


def _diag_body(q_ref, gt_ref):
    rio = lax.broadcasted_iota(jnp.int32, (B, B), 0)
    cio = lax.broadcasted_iota(jnp.int32, (B, B), 1)
    gt_ref[...] = jnp.sum(
        jnp.where(rio == cio, q_ref[...], 0.0), axis=0, keepdims=True
    )


def _diag(rowsq):
    return pl.pallas_call(
        _diag_body,
        out_shape=jax.ShapeDtypeStruct((1, B), jnp.float32),
    )(rowsq)


_CB = 2000
_NSTEP = C // _CB

_SHIFT = S * ((T_HARD + 1.0) + T_HARD)
_LOG2E = 1.4426950408889634
_K2 = S * _LOG2E
_M2 = _SHIFT * _LOG2E


def _ce_body(x_ref, g_ref, o_ref, acc):
    i = pl.program_id(0)

    @pl.when(i == 0)
    def _():
        acc[...] = jnp.zeros((1, B), jnp.float32)

    g = jnp.clip(g_ref[...], -1.0, 1.0)
    gm = g - M_MARGIN
    v = jnp.clip(x_ref[...], -1.0, 1.0)
    u = jnp.where(v > gm, (T_HARD + 1.0) * v + T_HARD, v)
    acc[...] += jnp.sum(jnp.exp2(u * _K2 - _M2), axis=0, keepdims=True)

    @pl.when(i == _NSTEP - 1)
    def _():
        fgt = jnp.where(g > 0.0, gm, g)
        trg = (T_HARD + 1.0) * g + T_HARD
        ssum = acc[...] - jnp.exp2(trg * _K2 - _M2) + jnp.exp2(fgt * _K2 - _M2)
        lse = jnp.log(ssum) + _SHIFT
        loss = jnp.sum(lse - S * fgt) * (1.0 / B)
        o_ref[...] = loss.reshape(1, 1)


def kernel(inputs, target):
    xt = inputs.T
    rowsq = _sc_rowgather(xt, target)
    gt = _diag(rowsq)
    loss = pl.pallas_call(
        _ce_body,
        grid=(_NSTEP,),
        in_specs=[
            pl.BlockSpec((_CB, B), lambda i: (i, 0)),
            pl.BlockSpec((1, B), lambda i: (0, 0)),
        ],
        out_specs=pl.BlockSpec((1, 1), lambda i: (0, 0)),
        out_shape=jax.ShapeDtypeStruct((1, 1), jnp.float32),
        scratch_shapes=[pltpu.VMEM((1, B), jnp.float32)],
    )(xt, gt)
    return loss[0, 0]
